# baseline (speedup 1.0000x reference)
import functools

import jax
import jax.numpy as jnp
from jax import lax
from jax.experimental import pallas as pl
from jax.experimental.pallas import tpu as pltpu
from jax.experimental.pallas import tpu_sc as plsc

N_CLASSES = 1000000
EMBED_DIM = 64
BATCH = 16384

_info = plsc.get_sparse_core_info()
_NC, _NS = _info.num_cores, _info.num_subcores
_NW = _NC * _NS
_B_PER_W = BATCH // _NW
_CHUNK = 128
_N_CHUNKS = _B_PER_W // _CHUNK


@functools.partial(
    pl.kernel,
    mesh=plsc.VectorSubcoreMesh(core_axis_name="c", subcore_axis_name="s"),
    out_type=jax.ShapeDtypeStruct((BATCH, EMBED_DIM), jnp.float32),
    scratch_types=[
        pltpu.VMEM((_NW, _N_CHUNKS, _CHUNK), jnp.int32),
        pltpu.VMEM((_B_PER_W, EMBED_DIM), jnp.float32),
        pltpu.SemaphoreType.DMA,
    ],
    compiler_params=pltpu.CompilerParams(use_tc_tiling_on_sc=False),
)
def _embed_gather(idx_hbm, table_hbm, out_hbm, idx_v, rows_v, sem):
    wid = lax.axis_index("s") * _NC + lax.axis_index("c")
    base = wid * _B_PER_W
    pltpu.sync_copy(idx_hbm.at[wid], idx_v.at[wid])
    copies = []
    for j in range(_N_CHUNKS):
        copies.append(
            pltpu.async_copy(
                table_hbm.at[idx_v.at[wid, j]],
                rows_v.at[pl.ds(j * _CHUNK, _CHUNK)],
                sem,
            )
        )
    for c in copies:
        c.wait()
    pltpu.sync_copy(rows_v, out_hbm.at[pl.ds(base, _B_PER_W)])


def kernel(class_labels, embedding_table):
    idx = class_labels.astype(jnp.int32).reshape(_NW, _N_CHUNKS, _CHUNK)
    out = _embed_gather(idx, embedding_table)
    return out.reshape(BATCH, 1, EMBED_DIM)


# speedup vs baseline: 1.4450x; 1.4450x over previous
import functools

import jax
import jax.numpy as jnp
from jax import lax
from jax.experimental import pallas as pl
from jax.experimental.pallas import tpu as pltpu

N_CLASSES = 1000000
EMBED_DIM = 64
BATCH = 16384

_RB = 512
_G = BATCH // _RB


def _gather_body(idx_sref, table_ref, out_ref, sem):
    g = pl.program_id(0)

    def issue(j, _):
        i = idx_sref[g * _RB + j]
        pltpu.make_async_copy(
            table_ref.at[pl.ds(i, 1), :],
            out_ref.at[pl.ds(j, 1), :],
            sem,
        ).start()
        return 0

    lax.fori_loop(0, _RB, issue, 0, unroll=8)
    pltpu.make_async_copy(
        table_ref.at[pl.ds(0, _RB), :],
        out_ref,
        sem,
    ).wait()


@jax.jit
def _embed_gather(labels, table):
    grid_spec = pltpu.PrefetchScalarGridSpec(
        num_scalar_prefetch=1,
        grid=(_G,),
        in_specs=[pl.BlockSpec(memory_space=pl.ANY)],
        out_specs=pl.BlockSpec((_RB, EMBED_DIM), lambda g, idx: (g, 0)),
        scratch_shapes=[pltpu.SemaphoreType.DMA],
    )
    return pl.pallas_call(
        _gather_body,
        grid_spec=grid_spec,
        out_shape=jax.ShapeDtypeStruct((BATCH, EMBED_DIM), jnp.float32),
    )(labels, table)


def kernel(class_labels, embedding_table):
    lab = class_labels.astype(jnp.int32)
    out = _embed_gather(lab, embedding_table)
    return out.reshape(BATCH, 1, EMBED_DIM)
